# Initial kernel scaffold; baseline (speedup 1.0000x reference)
import jax
import jax.numpy as jnp
from jax.experimental import pallas as pl


def kernel(x, edge_attr, em_w1, em_b1, em_w2, em_b2, c0_le_w, c0_le_b, c0_w1, c0_b1, c0_w2, c0_b2, c1_le_w, c1_le_b, c1_w1, c1_b1, c1_w2, c1_b2, edge_index, batch_idx):
    raise NotImplementedError("write your pallas kernel here")


# speedup vs baseline: 2.3654x; 2.3654x over previous
import functools

import jax
import jax.numpy as jnp
from jax import lax
from jax.experimental import pallas as pl
from jax.experimental.pallas import tpu as pltpu
from jax.experimental.pallas import tpu_sc as plsc

N = 10000
E = 320000
D_FEAT = 128
D_EDGE = 16
H = 128
G = 16

NC, NS = 2, 16
NW = NC * NS
EPW = E // NW
CH = 80
NCHUNK = EPW // CH
NP = 10240
RPT = NP // NS
ZB = 128

F32 = jnp.float32


def _edge_kernel(ea, w1, b1, w2, b2, lw0, lb0, lw1, lb1, p0, p1):
    t = jnp.maximum(ea[...] @ w1[...] + b1[...], 0.0)
    w0c = w2[...] @ lw0[...]
    w1c = w2[...] @ lw1[...]
    b0c = b2[...] @ lw0[...] + lb0[...]
    b1c = b2[...] @ lw1[...] + lb1[...]
    p0[...] = t @ w0c + b0c
    p1[...] = t @ w1c + b1c


def _edge_proj(ea, em_w1, em_b1, em_w2, em_b2, lw0, lb0, lw1, lb1):
    BE = 1280
    full = lambda i: (0, 0)
    return pl.pallas_call(
        _edge_kernel,
        grid=(E // BE,),
        in_specs=[
            pl.BlockSpec((BE, D_EDGE), lambda i: (i, 0)),
            pl.BlockSpec((D_EDGE, H), full),
            pl.BlockSpec((1, H), full),
            pl.BlockSpec((H, H), full),
            pl.BlockSpec((1, H), full),
            pl.BlockSpec((H, H), full),
            pl.BlockSpec((1, H), full),
            pl.BlockSpec((H, H), full),
            pl.BlockSpec((1, H), full),
        ],
        out_specs=[pl.BlockSpec((BE, H), lambda i: (i, 0))] * 2,
        out_shape=[jax.ShapeDtypeStruct((E, H), F32)] * 2,
        compiler_params=pltpu.CompilerParams(
            dimension_semantics=("arbitrary",)),
    )(ea, em_w1, em_b1.reshape(1, H), em_w2, em_b2.reshape(1, H),
      lw0, lb0.reshape(1, H), lw1, lb1.reshape(1, H))


def _sc_body(x_hbm, p_hbm, src_hbm, dst_hbm, out_hbm,
             src_v, dst_v, xg_v, pg_v, st_v, agg_sh, sem):
    cid = lax.axis_index("c")
    sid = lax.axis_index("s")
    wid = cid * NS + sid

    def _zrow(r, carry):
        for k in range(H // 16):
            st_v[r, pl.ds(k * 16, 16)] = jnp.zeros((16,), F32)
        return carry
    lax.fori_loop(0, ZB, _zrow, 0)

    def _zagg(j, carry):
        pltpu.sync_copy(st_v, agg_sh.at[pl.ds(sid * RPT + j * ZB, ZB)])
        return carry
    lax.fori_loop(0, RPT // ZB, _zagg, 0)
    plsc.subcore_barrier()

    base = wid * EPW

    def _chunk(i, carry):
        b = base + i * CH
        pltpu.sync_copy(src_hbm.at[pl.ds(b, CH)], src_v)
        pltpu.sync_copy(dst_hbm.at[pl.ds(b, CH)], dst_v)
        pltpu.sync_copy(p_hbm.at[pl.ds(b, CH)], pg_v)
        pltpu.async_copy(x_hbm.at[src_v], xg_v, sem).wait()

        def _row(r, c2):
            for k in range(H // 16):
                s = pl.ds(k * 16, 16)
                pg_v[r, s] = jnp.maximum(pg_v[r, s] + xg_v[r, s], 0.0)
            return c2
        lax.fori_loop(0, CH, _row, 0)
        pltpu.sync_copy(pg_v, agg_sh.at[dst_v], add=True)
        return carry
    lax.fori_loop(0, NCHUNK, _chunk, 0)
    plsc.subcore_barrier()

    def _wout(j, carry):
        rs = pl.ds(sid * RPT + j * ZB, ZB)
        pltpu.sync_copy(agg_sh.at[rs], st_v)
        pltpu.sync_copy(st_v, out_hbm.at[cid, rs])
        return carry
    lax.fori_loop(0, RPT // ZB, _wout, 0)


@functools.cache
def _get_sc_aggregate():
    mesh = plsc.VectorSubcoreMesh(
        core_axis_name="c", subcore_axis_name="s",
        num_cores=NC, num_subcores=NS)
    return pl.kernel(
        _sc_body,
        out_type=jax.ShapeDtypeStruct((NC, NP, H), F32),
        mesh=mesh,
        scratch_types=[
            pltpu.VMEM((CH,), jnp.int32),
            pltpu.VMEM((CH,), jnp.int32),
            pltpu.VMEM((CH, H), F32),
            pltpu.VMEM((CH, H), F32),
            pltpu.VMEM((ZB, H), F32),
            pltpu.VMEM_SHARED((NP, H), F32),
            pltpu.SemaphoreType.DMA,
        ],
    )


def _sc_aggregate(x, p, src, dst):
    return _get_sc_aggregate()(x, p, src, dst)


BN = 400


def _node_kernel(x, pa, pb, w1, b1, w2, b2, o):
    h = x[...] + pa[...] + pb[...]
    h = jnp.maximum(h @ w1[...] + b1[...], 0.0) @ w2[...] + b2[...]
    o[...] = jnp.maximum(h, 0.0)


def _node_mlp(x, pa, pb, w1, b1, w2, b2):
    full = lambda i: (0, 0)
    return pl.pallas_call(
        _node_kernel,
        grid=(N // BN,),
        in_specs=[
            pl.BlockSpec((BN, H), lambda i: (i, 0)),
            pl.BlockSpec((BN, H), lambda i: (i, 0)),
            pl.BlockSpec((BN, H), lambda i: (i, 0)),
            pl.BlockSpec((H, H), full),
            pl.BlockSpec((1, H), full),
            pl.BlockSpec((H, H), full),
            pl.BlockSpec((1, H), full),
        ],
        out_specs=pl.BlockSpec((BN, H), lambda i: (i, 0)),
        out_shape=jax.ShapeDtypeStruct((N, H), F32),
        compiler_params=pltpu.CompilerParams(
            dimension_semantics=("arbitrary",)),
    )(x, pa, pb, w1, b1.reshape(1, H), w2, b2.reshape(1, H))


def _final_kernel(x, pa, pb, w1, b1, w2, b2, bi, z, sums, cnts):
    i = pl.program_id(0)

    @pl.when(i == 0)
    def _init():
        sums[...] = jnp.zeros_like(sums)
        cnts[...] = jnp.zeros_like(cnts)

    h = x[...] + pa[...] + pb[...]
    h = jnp.maximum(h @ w1[...] + b1[...], 0.0) @ w2[...] + b2[...]
    h = jnp.maximum(h, 0.0)
    b = bi[0, 0, :]
    onehot = (b[:, None] == lax.broadcasted_iota(jnp.int32, (1, G), 1)
              ).astype(F32)
    sums[...] += lax.dot_general(onehot, h, (((0,), (0,)), ((), ())))
    cnt = jnp.sum(onehot, axis=0)
    cnts[...] += jnp.broadcast_to(cnt[:, None], (G, H))

    @pl.when(i == N // BN - 1)
    def _emit():
        z[...] = sums[...] / jnp.maximum(cnts[...], 1.0)


def _final_layer(x, pa, pb, w1, b1, w2, b2, batch_blocks):
    full = lambda i: (0, 0)
    return pl.pallas_call(
        _final_kernel,
        grid=(N // BN,),
        in_specs=[
            pl.BlockSpec((BN, H), lambda i: (i, 0)),
            pl.BlockSpec((BN, H), lambda i: (i, 0)),
            pl.BlockSpec((BN, H), lambda i: (i, 0)),
            pl.BlockSpec((H, H), full),
            pl.BlockSpec((1, H), full),
            pl.BlockSpec((H, H), full),
            pl.BlockSpec((1, H), full),
            pl.BlockSpec((1, 1, BN), lambda i: (i, 0, 0)),
        ],
        out_specs=pl.BlockSpec((G, H), full),
        out_shape=jax.ShapeDtypeStruct((G, H), F32),
        scratch_shapes=[pltpu.VMEM((G, H), F32), pltpu.VMEM((G, H), F32)],
        compiler_params=pltpu.CompilerParams(
            dimension_semantics=("arbitrary",)),
    )(x, pa, pb, w1, b1.reshape(1, H), w2, b2.reshape(1, H), batch_blocks)


def kernel(x, edge_attr, em_w1, em_b1, em_w2, em_b2,
           c0_le_w, c0_le_b, c0_w1, c0_b1, c0_w2, c0_b2,
           c1_le_w, c1_le_b, c1_w1, c1_b1, c1_w2, c1_b2,
           edge_index, batch_idx):
    src = edge_index[0]
    dst = edge_index[1]

    p0, p1 = _edge_proj(edge_attr, em_w1, em_b1, em_w2, em_b2,
                        c0_le_w, c0_le_b, c1_le_w, c1_le_b)

    agg0 = _sc_aggregate(x, p0, src, dst)
    h0 = _node_mlp(x, agg0[0], agg0[1], c0_w1, c0_b1, c0_w2, c0_b2)

    agg1 = _sc_aggregate(h0, p1, src, dst)
    z = _final_layer(h0, agg1[0], agg1[1], c1_w1, c1_b1, c1_w2, c1_b2,
                     batch_idx.reshape(N // BN, 1, BN))
    return z


# speedup vs baseline: 3.7928x; 1.6035x over previous
import functools

import jax
import jax.numpy as jnp
from jax import lax
from jax.experimental import pallas as pl
from jax.experimental.pallas import tpu as pltpu
from jax.experimental.pallas import tpu_sc as plsc

N = 10000
E = 320000
D_FEAT = 128
D_EDGE = 16
H = 128
G = 16

NC, NS = 2, 16
NW = NC * NS
EPW = E // NW
CH = 80
NCHUNK = EPW // CH
NP = 10240
RPT = NP // NS
ZB = 128

F32 = jnp.float32


def _edge_kernel(ea, w1, b1, w2, b2, lw0, lb0, lw1, lb1, p0, p1):
    t = jnp.maximum(ea[...] @ w1[...] + b1[...], 0.0)
    w0c = w2[...] @ lw0[...]
    w1c = w2[...] @ lw1[...]
    b0c = b2[...] @ lw0[...] + lb0[...]
    b1c = b2[...] @ lw1[...] + lb1[...]
    p0[...] = t @ w0c + b0c
    p1[...] = t @ w1c + b1c


def _edge_proj(ea, em_w1, em_b1, em_w2, em_b2, lw0, lb0, lw1, lb1):
    BE = 1280
    full = lambda i: (0, 0)
    return pl.pallas_call(
        _edge_kernel,
        grid=(E // BE,),
        in_specs=[
            pl.BlockSpec((BE, D_EDGE), lambda i: (i, 0)),
            pl.BlockSpec((D_EDGE, H), full),
            pl.BlockSpec((1, H), full),
            pl.BlockSpec((H, H), full),
            pl.BlockSpec((1, H), full),
            pl.BlockSpec((H, H), full),
            pl.BlockSpec((1, H), full),
            pl.BlockSpec((H, H), full),
            pl.BlockSpec((1, H), full),
        ],
        out_specs=[pl.BlockSpec((BE, H), lambda i: (i, 0))] * 2,
        out_shape=[jax.ShapeDtypeStruct((E, H), F32)] * 2,
        compiler_params=pltpu.CompilerParams(
            dimension_semantics=("arbitrary",)),
    )(ea, em_w1, em_b1.reshape(1, H), em_w2, em_b2.reshape(1, H),
      lw0, lb0.reshape(1, H), lw1, lb1.reshape(1, H))


def _sc_body(x_hbm, p_hbm, src_hbm, dst_hbm, out_hbm,
             src0, src1, dst0, dst1, pg0, pg1, xg0, st_v, agg_sh,
             ssrc0, ssrc1, sdst0, sdst1, sp0, sp1, sg0, ssc0, ssc1):
    SRC = (src0, src1)
    DST = (dst0, dst1)
    PG = (pg0, pg1)
    XG = (xg0, xg0)
    SSRC = (ssrc0, ssrc1)
    SDST = (sdst0, sdst1)
    SP = (sp0, sp1)
    SG = (sg0, sg0)
    SSC = (ssc0, ssc1)

    cid = lax.axis_index("c")
    sid = lax.axis_index("s")
    wid = cid * NS + sid
    base = wid * EPW

    def _zrow(r, carry):
        for k in range(H // 16):
            st_v[r, pl.ds(k * 16, 16)] = jnp.zeros((16,), F32)
        return carry
    lax.fori_loop(0, ZB, _zrow, 0)

    def _zagg(j, carry):
        pltpu.sync_copy(st_v, agg_sh.at[pl.ds(sid * RPT + j * ZB, ZB)])
        return carry
    lax.fori_loop(0, RPT // ZB, _zagg, 0)
    plsc.subcore_barrier()

    def _issue_loads(j, b):
        bb = jnp.minimum(base + j * CH, E - CH)
        pltpu.async_copy(src_hbm.at[pl.ds(bb, CH)], SRC[b], SSRC[b])
        pltpu.async_copy(dst_hbm.at[pl.ds(bb, CH)], DST[b], SDST[b])
        pltpu.async_copy(p_hbm.at[pl.ds(bb, CH)], PG[b], SP[b])

    def _wait_src(b):
        pltpu.make_async_copy(src_hbm.at[pl.ds(0, CH)], SRC[b], SSRC[b]).wait()

    def _wait_dstp(b):
        pltpu.make_async_copy(dst_hbm.at[pl.ds(0, CH)], DST[b], SDST[b]).wait()
        pltpu.make_async_copy(p_hbm.at[pl.ds(0, CH)], PG[b], SP[b]).wait()

    def _wait_scat(b):
        pltpu.make_async_copy(PG[b], agg_sh.at[DST[b]], SSC[b]).wait()

    def _compute(b):
        pg, xg = PG[b], XG[b]

        def _row(r, c2):
            for k in range(H // 16):
                s = pl.ds(k * 16, 16)
                pg[r, s] = jnp.maximum(pg[r, s] + xg[r, s], 0.0)
            return c2
        lax.fori_loop(0, CH, _row, 0)

    def _section(j, b, first):
        o = 1 - b
        _wait_src(b)
        g = pltpu.async_copy(x_hbm.at[SRC[b]], XG[b], SG[b])
        if not first:
            _wait_scat(o)
        _issue_loads(j + 1, o)
        _wait_dstp(b)
        g.wait()
        _compute(b)
        pltpu.async_copy(PG[b], agg_sh.at[DST[b]], SSC[b], add=True)

    _issue_loads(0, 0)
    _section(0, 0, True)

    def _pair(k, carry):
        _section(2 * k + 1, 1, False)
        _section(2 * k + 2, 0, False)
        return carry
    lax.fori_loop(0, (NCHUNK - 1) // 2, _pair, 0)

    _wait_scat(0)
    _wait_src(1)
    _wait_dstp(1)
    plsc.subcore_barrier()

    def _wout(j, carry):
        rs = pl.ds(sid * RPT + j * ZB, ZB)
        pltpu.sync_copy(agg_sh.at[rs], st_v)
        pltpu.sync_copy(st_v, out_hbm.at[cid, rs])
        return carry
    lax.fori_loop(0, RPT // ZB, _wout, 0)


@functools.cache
def _get_sc_aggregate():
    mesh = plsc.VectorSubcoreMesh(
        core_axis_name="c", subcore_axis_name="s",
        num_cores=NC, num_subcores=NS)
    return pl.kernel(
        _sc_body,
        out_type=jax.ShapeDtypeStruct((NC, NP, H), F32),
        mesh=mesh,
        scratch_types=(
            [pltpu.VMEM((CH,), jnp.int32)] * 4
            + [pltpu.VMEM((CH, H), F32)] * 3
            + [pltpu.VMEM((ZB, H), F32)]
            + [pltpu.VMEM_SHARED((NP, H), F32)]
            + [pltpu.SemaphoreType.DMA] * 9
        ),
    )


def _sc_aggregate(x, p, src, dst):
    return _get_sc_aggregate()(x, p, src, dst)


BN = 400


def _node_kernel(x, pa, pb, w1, b1, w2, b2, o):
    h = x[...] + pa[...] + pb[...]
    h = jnp.maximum(h @ w1[...] + b1[...], 0.0) @ w2[...] + b2[...]
    o[...] = jnp.maximum(h, 0.0)


def _node_mlp(x, pa, pb, w1, b1, w2, b2):
    full = lambda i: (0, 0)
    return pl.pallas_call(
        _node_kernel,
        grid=(N // BN,),
        in_specs=[
            pl.BlockSpec((BN, H), lambda i: (i, 0)),
            pl.BlockSpec((BN, H), lambda i: (i, 0)),
            pl.BlockSpec((BN, H), lambda i: (i, 0)),
            pl.BlockSpec((H, H), full),
            pl.BlockSpec((1, H), full),
            pl.BlockSpec((H, H), full),
            pl.BlockSpec((1, H), full),
        ],
        out_specs=pl.BlockSpec((BN, H), lambda i: (i, 0)),
        out_shape=jax.ShapeDtypeStruct((N, H), F32),
        compiler_params=pltpu.CompilerParams(
            dimension_semantics=("arbitrary",)),
    )(x, pa, pb, w1, b1.reshape(1, H), w2, b2.reshape(1, H))


def _final_kernel(x, pa, pb, w1, b1, w2, b2, bi, z, sums, cnts):
    i = pl.program_id(0)

    @pl.when(i == 0)
    def _init():
        sums[...] = jnp.zeros_like(sums)
        cnts[...] = jnp.zeros_like(cnts)

    h = x[...] + pa[...] + pb[...]
    h = jnp.maximum(h @ w1[...] + b1[...], 0.0) @ w2[...] + b2[...]
    h = jnp.maximum(h, 0.0)
    b = bi[0, 0, :]
    onehot = (b[:, None] == lax.broadcasted_iota(jnp.int32, (1, G), 1)
              ).astype(F32)
    sums[...] += lax.dot_general(onehot, h, (((0,), (0,)), ((), ())))
    cnt = jnp.sum(onehot, axis=0)
    cnts[...] += jnp.broadcast_to(cnt[:, None], (G, H))

    @pl.when(i == N // BN - 1)
    def _emit():
        z[...] = sums[...] / jnp.maximum(cnts[...], 1.0)


def _final_layer(x, pa, pb, w1, b1, w2, b2, batch_blocks):
    full = lambda i: (0, 0)
    return pl.pallas_call(
        _final_kernel,
        grid=(N // BN,),
        in_specs=[
            pl.BlockSpec((BN, H), lambda i: (i, 0)),
            pl.BlockSpec((BN, H), lambda i: (i, 0)),
            pl.BlockSpec((BN, H), lambda i: (i, 0)),
            pl.BlockSpec((H, H), full),
            pl.BlockSpec((1, H), full),
            pl.BlockSpec((H, H), full),
            pl.BlockSpec((1, H), full),
            pl.BlockSpec((1, 1, BN), lambda i: (i, 0, 0)),
        ],
        out_specs=pl.BlockSpec((G, H), full),
        out_shape=jax.ShapeDtypeStruct((G, H), F32),
        scratch_shapes=[pltpu.VMEM((G, H), F32), pltpu.VMEM((G, H), F32)],
        compiler_params=pltpu.CompilerParams(
            dimension_semantics=("arbitrary",)),
    )(x, pa, pb, w1, b1.reshape(1, H), w2, b2.reshape(1, H), batch_blocks)


def kernel(x, edge_attr, em_w1, em_b1, em_w2, em_b2,
           c0_le_w, c0_le_b, c0_w1, c0_b1, c0_w2, c0_b2,
           c1_le_w, c1_le_b, c1_w1, c1_b1, c1_w2, c1_b2,
           edge_index, batch_idx):
    src = edge_index[0]
    dst = edge_index[1]

    p0, p1 = _edge_proj(edge_attr, em_w1, em_b1, em_w2, em_b2,
                        c0_le_w, c0_le_b, c1_le_w, c1_le_b)

    agg0 = _sc_aggregate(x, p0, src, dst)
    h0 = _node_mlp(x, agg0[0], agg0[1], c0_w1, c0_b1, c0_w2, c0_b2)

    agg1 = _sc_aggregate(h0, p1, src, dst)
    z = _final_layer(h0, agg1[0], agg1[1], c1_w1, c1_b1, c1_w2, c1_b2,
                     batch_idx.reshape(N // BN, 1, BN))
    return z
